# baseline (speedup 1.0000x reference)
import functools
import math

import jax
import jax.numpy as jnp
from jax import lax
from jax.experimental import pallas as pl
from jax.experimental.pallas import tpu as pltpu
from jax.experimental.pallas import tpu_sc as plsc

_VOCAB = 1000000
_EMBED = 128
_BATCH = 4096
_HIST = 200
_B = _BATCH * _HIST

_NC = 2
_NS = 16
_NW = _NC * _NS
_B_PER_W = _B // _NW
_CHUNK = 40
_NBUF = 8
_NSLOT = 4
_NCHUNK = _B_PER_W // _CHUNK
_NROUND = _NCHUNK // _NBUF
_LANES = 16
_SCALE = float(math.sqrt(float(_EMBED)))

_mesh = plsc.VectorSubcoreMesh(core_axis_name="c", subcore_axis_name="s")


@functools.partial(
    pl.kernel,
    mesh=_mesh,
    out_type=jax.ShapeDtypeStruct((_B, _EMBED), jnp.float32),
    scratch_types=(
        [pltpu.VMEM((_B_PER_W,), jnp.int32)]
        + [pltpu.VMEM((_CHUNK, _EMBED), jnp.float32)] * (2 * _NBUF)
        + [pltpu.SemaphoreType.DMA] * (2 * _NBUF + _NSLOT)
        + [pltpu.VMEM_SHARED((_NS, _NSLOT, _CHUNK, _EMBED), jnp.float32)]
    ),
)
def _embed_sc(idx_hbm, table_hbm, out_hbm, idx_v, *bufs_and_sems):
    ins = bufs_and_sems[:_NBUF]
    outs = bufs_and_sems[_NBUF:2 * _NBUF]
    gsems = bufs_and_sems[2 * _NBUF:3 * _NBUF]
    ssems = bufs_and_sems[3 * _NBUF:4 * _NBUF]
    dsems = bufs_and_sems[4 * _NBUF:4 * _NBUF + _NSLOT]
    shared = bufs_and_sems[4 * _NBUF + _NSLOT]
    sid = lax.axis_index("s")

    wid = lax.axis_index("s") * _NC + lax.axis_index("c")
    base = wid * _B_PER_W
    pltpu.sync_copy(idx_hbm.at[pl.ds(base, _B_PER_W)], idx_v)

    for b in range(_NBUF):
        pltpu.async_copy(
            table_hbm.at[idx_v.at[pl.ds(b * _CHUNK, _CHUNK)]], ins[b], gsems[b]
        )

    def round_body(h, carry):
        for b in range(_NBUF):
            g = _NBUF * h + b
            t = b % _NSLOT
            bp = (b - 2) % _NBUF
            tp = (b - 2) % _NSLOT
            inb, outb, gsb = ins[b], outs[b], gsems[b]
            pltpu.make_async_copy(
                table_hbm.at[idx_v.at[pl.ds(g * _CHUNK, _CHUNK)]], inb, gsb
            ).wait()

            def row_body(i, c):
                for j in range(_EMBED // _LANES):
                    sl = (i, pl.ds(j * _LANES, _LANES))
                    outb[sl] = inb[sl] * _SCALE
                return c

            lax.fori_loop(0, _CHUNK, row_body, 0)

            @pl.when(g >= _NSLOT)
            def _wait_slot_free():
                pltpu.make_async_copy(
                    shared.at[sid, t],
                    out_hbm.at[pl.ds(base + (g - _NSLOT) * _CHUNK, _CHUNK)],
                    dsems[t],
                ).wait()

            pltpu.async_copy(outb, shared.at[sid, t], ssems[b])

            @pl.when(g >= 2)
            def _drain_prev():
                pltpu.make_async_copy(
                    outs[bp], shared.at[sid, tp], ssems[bp]
                ).wait()
                pltpu.async_copy(
                    shared.at[sid, tp],
                    out_hbm.at[pl.ds(base + (g - 2) * _CHUNK, _CHUNK)],
                    dsems[tp],
                )

            @pl.when(g + _NBUF < _NCHUNK)
            def _next_gather():
                pltpu.async_copy(
                    table_hbm.at[idx_v.at[pl.ds((g + _NBUF) * _CHUNK, _CHUNK)]],
                    inb, gsb,
                )

        return carry

    lax.fori_loop(0, _NROUND, round_body, 0)

    for p in (_NCHUNK - 2, _NCHUNK - 1):
        b = p % _NBUF
        t = p % _NSLOT
        pltpu.make_async_copy(outs[b], shared.at[sid, t], ssems[b]).wait()
        pltpu.async_copy(
            shared.at[sid, t],
            out_hbm.at[pl.ds(base + p * _CHUNK, _CHUNK)],
            dsems[t],
        )
    for p in range(_NCHUNK - _NSLOT, _NCHUNK):
        t = p % _NSLOT
        pltpu.make_async_copy(
            shared.at[sid, t],
            out_hbm.at[pl.ds(base + p * _CHUNK, _CHUNK)],
            dsems[t],
        ).wait()


def kernel(tokens, table):
    idx = tokens.reshape(-1)
    out = _embed_sc(idx, table)
    return out.reshape(_BATCH, _HIST, _EMBED)
